# baseline (speedup 1.0000x reference)
import functools

import jax
import jax.numpy as jnp
from jax import lax
from jax.experimental import pallas as pl
from jax.experimental.pallas import tpu as pltpu
from jax.experimental.pallas import tpu_sc as plsc

D_MODEL = 128
N_ROWS = 16384 * 200
NC, NS, L = 2, 16, 16
NW = NC * NS
ROWS_PER_W = N_ROWS // NW
CHUNK = 512
K = CHUNK // 128
N_CHUNKS = ROWS_PER_W // CHUNK


def _sc_lookup(x_hbm, table_hbm, out_hbm,
               x0, x1, x2, x3, fstage, i0, i1, i2, i3, rowbuf, sem):
    wid = lax.axis_index("s") * NC + lax.axis_index("c")
    w_base = wid * ROWS_PER_W
    xidx = [x0, x1, x2, x3]
    idxbufs = [i0, i1, i2, i3]

    def chunk_body(g, carry):
        base = w_base + g * CHUNK
        lane8 = lax.iota(jnp.int32, L) * 8
        for j in range(K):
            for m in range(8):
                off = (base + j * 128 + m * L) * 8 + 7
                xidx[j][pl.ds(m * L, L)] = lane8 + off
        hx = [
            pltpu.async_copy(x_hbm.at[xidx[j]],
                             fstage.at[pl.ds(j * 128, 128)], sem)
            for j in range(K)
        ]
        for h in hx:
            h.wait()
        for i in range(CHUNK // L):
            v = fstage[pl.ds(i * L, L)]
            idxbufs[i // 8][pl.ds((i % 8) * L, L)] = v.astype(jnp.int32)
        ht = [
            pltpu.async_copy(table_hbm.at[idxbufs[j]],
                             rowbuf.at[pl.ds(j * 128, 128)], sem)
            for j in range(K)
        ]
        for h in ht:
            h.wait()
        pltpu.sync_copy(rowbuf, out_hbm.at[pl.ds(base, CHUNK)])
        return carry

    lax.fori_loop(0, N_CHUNKS, chunk_body, 0)


def kernel(x, table):
    x_flat = x.reshape(N_ROWS * 8)
    mesh = plsc.VectorSubcoreMesh(core_axis_name="c", subcore_axis_name="s")
    f = functools.partial(
        pl.kernel,
        mesh=mesh,
        out_type=jax.ShapeDtypeStruct((N_ROWS, D_MODEL), jnp.float32),
        scratch_types=[
            pltpu.VMEM((128,), jnp.int32),
            pltpu.VMEM((128,), jnp.int32),
            pltpu.VMEM((128,), jnp.int32),
            pltpu.VMEM((128,), jnp.int32),
            pltpu.VMEM((CHUNK,), jnp.float32),
            pltpu.VMEM((128,), jnp.int32),
            pltpu.VMEM((128,), jnp.int32),
            pltpu.VMEM((128,), jnp.int32),
            pltpu.VMEM((128,), jnp.int32),
            pltpu.VMEM((CHUNK, D_MODEL), jnp.float32),
            pltpu.SemaphoreType.DMA,
        ],
    )(_sc_lookup)
    out = f(x_flat, table)
    return out.reshape(x.shape[0], x.shape[1], D_MODEL)
